# baseline (speedup 1.0000x reference)
import functools
import jax
import jax.numpy as jnp
from jax import lax
from jax.experimental import pallas as pl
from jax.experimental.pallas import tpu as pltpu, tpu_sc as plsc

N = 262144
EPS = 1e-7
NT = 16
CHUNK = N // NT
B = 32768
SLICE = B // NT
W = 2048
NW = CHUNK // W
NBATCH = 512
NDB = SLICE // NBATCH
CAP = 6144
CAPBUF = CAP + 32
IDXMASK = (1 << 18) - 1
LN2 = 0.6931471805599453


def _log16(x):
    xi = lax.bitcast_convert_type(x, jnp.int32)
    e = ((xi >> 23) & 0xFF) - 127
    m = lax.bitcast_convert_type((xi & 0x7FFFFF) | 0x3F800000, jnp.float32)
    big = m > 1.4142135
    r = jnp.where(big, m * 0.5, m)
    ef = (e + jnp.where(big, 1, 0)).astype(jnp.float32)
    t = (r - 1.0) / (r + 1.0)
    t2 = t * t
    lnr = 2.0 * t * (1.0 + t2 * (1.0 / 3.0 + t2 * (0.2 + t2 / 7.0)))
    return ef * LN2 + lnr


def _dgather(v, idx):
    return lax.gather(
        v, idx[:, None],
        dimension_numbers=lax.GatherDimensionNumbers(
            offset_dims=(), collapsed_slice_dims=(0,), start_index_map=(0,)),
        slice_sizes=(1,),
        mode=lax.GatherScatterMode.PROMISE_IN_BOUNDS)


def kernel(input, target, weight):
    mesh = plsc.VectorSubcoreMesh(
        core_axis_name="c", subcore_axis_name="s", num_cores=1)

    @functools.partial(
        pl.kernel, mesh=mesh,
        out_type=jax.ShapeDtypeStruct((16,), jnp.float32),
        scratch_types=[
            pltpu.VMEM((B,), jnp.int32),
            pltpu.VMEM((W,), jnp.float32),
            pltpu.VMEM((W,), jnp.float32),
            pltpu.VMEM((W,), jnp.float32),
            pltpu.VMEM((W,), jnp.int32),
            pltpu.VMEM((W,), jnp.float32),
            pltpu.VMEM((W,), jnp.int32),
            pltpu.VMEM((W,), jnp.int32),
            pltpu.VMEM((W,), jnp.int32),
            pltpu.VMEM((CAPBUF,), jnp.float32),
            pltpu.VMEM((CAPBUF,), jnp.int32),
            pltpu.VMEM((CAPBUF,), jnp.float32),
            pltpu.VMEM((SLICE,), jnp.int32),
            pltpu.VMEM((SLICE,), jnp.float32),
            pltpu.VMEM((256,), jnp.float32),
            pltpu.VMEM((256,), jnp.int32),
            pltpu.VMEM((NBATCH + 16,), jnp.int32),
            pltpu.SMEM((NBATCH,), jnp.float32),
            pltpu.SemaphoreType.DMA,
            pltpu.SemaphoreType.DMA,
            pltpu.VMEM_SHARED((NT * B,), jnp.int32),
            pltpu.HBM((B + 16,), jnp.int32),
            pltpu.VMEM_SHARED((B,), jnp.float32),
            pltpu.HBM((N,), jnp.float32),
            pltpu.HBM((N,), jnp.int32),
            pltpu.HBM((N,), jnp.float32),
            pltpu.VMEM_SHARED((NT * 16,), jnp.float32),
            pltpu.VMEM_SHARED((NT * 16,), jnp.float32),
            pltpu.VMEM_SHARED((NT * 16,), jnp.float32),
            pltpu.VMEM_SHARED((NT * 16,), jnp.float32),
            pltpu.VMEM_SHARED((NT * 16,), jnp.int32),
        ],
        compiler_params=pltpu.CompilerParams(needs_layout_passes=False),
    )
    def k(lh_hbm, d_hbm, ev_hbm, out_hbm,
          v_cnt, wd, wlh, wev, wpk, wex, wpos, wbeta, wone,
          bd, bpk, bex, v_sl, v_fl, v_tf, v_ti, vbst, smf, sem, sem_in,
          sh_hist, sh_bstart, sh_bsum, sh_gd, sh_gpk, sh_gex,
          sh_max, sh_se, sh_selh, sh_T, sh_tot):
        tid = lax.axis_index("s")
        iota = lax.iota(jnp.int32, 16)
        zf = jnp.zeros((16,), jnp.float32)
        zi = jnp.zeros((16,), jnp.int32)
        full15 = jnp.full((16,), 15, jnp.int32)

        def stage3(base):
            pltpu.sync_copy(d_hbm.at[pl.ds(base, W)], wd)
            pltpu.sync_copy(lh_hbm.at[pl.ds(base, W)], wlh)
            pltpu.sync_copy(ev_hbm.at[pl.ds(base, W)], wev)

        def _zero_cnt(i, _):
            v_cnt[pl.ds(i * 16, 16)] = zi
            return 0
        lax.fori_loop(0, B // 16, _zero_cnt, 0)

        def _zero_fl(i, _):
            v_fl[pl.ds(i * 16, 16)] = zf
            return 0
        lax.fori_loop(0, SLICE // 16, _zero_fl, 0)
        pltpu.sync_copy(v_fl, sh_bsum.at[pl.ds(tid * SLICE, SLICE)])

        def _init_hist(r, _):
            pltpu.sync_copy(v_cnt.at[pl.ds(r * SLICE, SLICE)],
                            sh_hist.at[pl.ds(tid * B + r * SLICE, SLICE)])
            return 0
        lax.fori_loop(0, NT, _init_hist, 0)
        plsc.subcore_barrier()

        histbase = tid * B

        def pa_win(w, carry):
            mx, se, selh = carry
            stage3(tid * CHUNK + w * W)

            def body(i, c):
                mx, se, selh = c
                d = wd[pl.ds(i * 16, 16)]
                lh = wlh[pl.ds(i * 16, 16)]
                ev = wev[pl.ds(i * 16, 16)]
                beta = (B - 1) - jnp.minimum(
                    (d * (B / 100.0)).astype(jnp.int32), B - 1)
                wbeta[pl.ds(i * 16, 16)] = beta + histbase
                wone[pl.ds(i * 16, 16)] = jnp.ones((16,), jnp.int32)
                return (jnp.maximum(mx, lh), se + ev, selh + ev * lh)
            mx, se, selh = lax.fori_loop(0, W // 16, body, (mx, se, selh))
            pltpu.sync_copy(wone, sh_hist.at[wbeta], add=True)
            return (mx, se, selh)

        mx, se, selh = lax.fori_loop(
            0, NW, pa_win,
            (jnp.full((16,), -3.4e38, jnp.float32), zf, zf))

        v_tf[pl.ds(0, 16)] = mx
        pltpu.sync_copy(v_tf.at[pl.ds(0, 16)], sh_max.at[pl.ds(tid * 16, 16)])
        v_tf[pl.ds(0, 16)] = se
        pltpu.sync_copy(v_tf.at[pl.ds(0, 16)], sh_se.at[pl.ds(tid * 16, 16)])
        v_tf[pl.ds(0, 16)] = selh
        pltpu.sync_copy(v_tf.at[pl.ds(0, 16)], sh_selh.at[pl.ds(tid * 16, 16)])
        plsc.subcore_barrier()

        pltpu.sync_copy(sh_max, v_tf.at[pl.ds(0, 256)])

        def _gmax(r, g):
            return jnp.maximum(g, v_tf[pl.ds(r * 16, 16)])
        gmx = lax.fori_loop(0, NT, _gmax,
                            jnp.full((16,), -3.4e38, jnp.float32))
        gamma = jnp.max(gmx)

        def _stage_cols(r, _):
            pltpu.sync_copy(sh_hist.at[pl.ds(r * B + tid * SLICE, SLICE)],
                            v_cnt.at[pl.ds(r * SLICE, SLICE)])
            return 0
        lax.fori_loop(0, NT, _stage_cols, 0)

        def _tot(i, _):
            acc = zi
            for r in range(NT):
                acc = acc + v_cnt[pl.ds(r * SLICE + i * 16, 16)]
            v_sl[pl.ds(i * 16, 16)] = acc
            return 0
        lax.fori_loop(0, SLICE // 16, _tot, 0)

        def _scan_i(i, carry):
            v = v_sl[pl.ds(i * 16, 16)]
            cs = plsc.cumsum(v) + carry
            v_sl[pl.ds(i * 16, 16)] = cs
            return _dgather(cs, full15)
        carry = lax.fori_loop(0, SLICE // 16, _scan_i, zi)
        v_ti[pl.ds(0, 16)] = carry
        pltpu.sync_copy(v_ti.at[pl.ds(0, 16)], sh_tot.at[pl.ds(tid * 16, 16)])
        plsc.subcore_barrier()

        pltpu.sync_copy(sh_tot, v_ti.at[pl.ds(0, 256)])
        tot16 = plsc.load_gather(v_ti, [iota * 16])
        exc = plsc.cumsum(tot16) - tot16
        mycarry = _dgather(exc, jnp.full((16,), tid, jnp.int32))

        def _excl(i, _):
            incl = v_sl[pl.ds(i * 16, 16)]
            tot = zi
            for r in range(NT):
                tot = tot + v_cnt[pl.ds(r * SLICE + i * 16, 16)]
            v_sl[pl.ds(i * 16, 16)] = incl - tot + mycarry
            return 0
        lax.fori_loop(0, SLICE // 16, _excl, 0)
        pltpu.sync_copy(v_sl, sh_bstart.at[pl.ds(tid * SLICE, SLICE)])

        @pl.when(tid == NT - 1)
        def _sentinel():
            v_ti[pl.ds(16, 16)] = jnp.full((16,), N, jnp.int32)
            pltpu.sync_copy(v_ti.at[pl.ds(16, 16)], sh_bstart.at[pl.ds(B, 16)])

        def _bases(r, _):
            def bl(i, _):
                acc = v_sl[pl.ds(i * 16, 16)]
                h = v_cnt[pl.ds(r * SLICE + i * 16, 16)]
                v_sl[pl.ds(i * 16, 16)] = acc + h
                v_cnt[pl.ds(r * SLICE + i * 16, 16)] = acc
                return 0
            lax.fori_loop(0, SLICE // 16, bl, 0)
            pltpu.sync_copy(v_cnt.at[pl.ds(r * SLICE, SLICE)],
                            sh_hist.at[pl.ds(r * B + tid * SLICE, SLICE)])
            return 0
        lax.fori_loop(0, NT, _bases, 0)
        plsc.subcore_barrier()

        pltpu.sync_copy(sh_hist.at[pl.ds(tid * B, B)], v_cnt)
        plsc.subcore_barrier()

        def pc_win(w, _):
            cbase = tid * CHUNK + w * W
            stage3(cbase)

            def body(i, _):
                j = i * 16
                d = wd[pl.ds(j, 16)]
                lh = wlh[pl.ds(j, 16)]
                ev = wev[pl.ds(j, 16)]
                beta = (B - 1) - jnp.minimum(
                    (d * (B / 100.0)).astype(jnp.int32), B - 1)
                ex = jnp.exp(lh - gamma)
                gidx = cbase + j + iota
                pk = gidx | (ev.astype(jnp.int32) << 30)
                wpos[pl.ds(j, 16)] = cbase + j + iota
                wbeta[pl.ds(j, 16)] = beta
                wpk[pl.ds(j, 16)] = pk
                wex[pl.ds(j, 16)] = ex
                return 0
            lax.fori_loop(0, W // 16, body, 0)
            pltpu.sync_copy(wd, sh_gd.at[wpos])
            pltpu.sync_copy(wpk, sh_gpk.at[wpos])
            pltpu.sync_copy(wex, sh_gex.at[wpos])
            pltpu.sync_copy(wex, sh_bsum.at[wbeta], add=True)
            return 0
        lax.fori_loop(0, NW, pc_win, 0)
        plsc.subcore_barrier()

        pltpu.sync_copy(sh_bsum.at[pl.ds(tid * SLICE, SLICE)], v_fl)

        def _fscan(i, carry):
            v = v_fl[pl.ds(i * 16, 16)]
            cs = plsc.cumsum(v) + carry
            v_fl[pl.ds(i * 16, 16)] = cs - v
            return _dgather(cs, full15)
        fcarry = lax.fori_loop(0, SLICE // 16, _fscan, zf)
        v_tf[pl.ds(0, 16)] = fcarry
        pltpu.sync_copy(v_tf.at[pl.ds(0, 16)], sh_T.at[pl.ds(tid * 16, 16)])
        plsc.subcore_barrier()
        pltpu.sync_copy(sh_T, v_tf.at[pl.ds(0, 256)])
        ftot16 = plsc.load_gather(v_tf, [iota * 16])
        fexc = plsc.cumsum(ftot16) - ftot16
        myf = _dgather(fexc, jnp.full((16,), tid, jnp.int32))

        def _fadd(i, _):
            v_fl[pl.ds(i * 16, 16)] = v_fl[pl.ds(i * 16, 16)] + myf
            return 0
        lax.fori_loop(0, SLICE // 16, _fadd, 0)
        pltpu.sync_copy(v_fl, sh_bsum.at[pl.ds(tid * SLICE, SLICE)])
        plsc.subcore_barrier()

        def pd_batch(db, accT):
            bb = tid * SLICE + db * NBATCH
            pltpu.sync_copy(sh_bstart.at[pl.ds(bb, NBATCH + 16)], vbst)
            pltpu.sync_copy(sh_bsum.at[pl.ds(bb, NBATCH)], smf)
            s0 = vbst[pl.ds(0, 16)][0]
            s8 = (s0 // 8) * 8
            start = jnp.minimum(s8, N - CAP)
            pltpu.sync_copy(sh_gd.at[pl.ds(start, CAP)], bd.at[pl.ds(0, CAP)])
            pltpu.sync_copy(sh_gpk.at[pl.ds(start, CAP)],
                            bpk.at[pl.ds(0, CAP)])
            pltpu.sync_copy(sh_gex.at[pl.ds(start, CAP)],
                            bex.at[pl.ds(0, CAP)])

            def bucket_body(b, accT):
                b2 = vbst[pl.ds(b, 16)]
                sb_ = b2[0]
                kk = b2[1] - sb_
                off = sb_ - start
                Gb = smf[b]

                def fast(accT):
                    d16 = bd[pl.ds(off, 16)]
                    pk16 = bpk[pl.ds(off, 16)]
                    ex16 = bex[pl.ds(off, 16)]
                    lmask = iota < kk
                    sd, sv, _m1 = plsc.sort_key_val(
                        d16, iota, mask=lmask, descending=True)
                    svc = jnp.where(lmask, sv, 0)
                    spk = _dgather(pk16, svc)
                    sex = _dgather(ex16, svc)
                    prev = _dgather(sd, jnp.maximum(iota - 1, 0))
                    eqp = jnp.logical_and(sd == prev, iota > 0)
                    rstart = plsc.cummax(jnp.where(eqp, 0, iota))
                    key2 = (rstart << 18) | (spk & IDXMASK)
                    key2 = jnp.where(lmask, key2, 0x7FFFFFFF)
                    _k2, sv2 = plsc.sort_key_val(key2, iota)
                    pkf = _dgather(spk, sv2)
                    exf = _dgather(sex, sv2)
                    exm = jnp.where(lmask, exf, 0.0)
                    S = plsc.cumsum(exm) + Gb
                    lg = _log16(S + EPS)
                    evf = ((pkf >> 30) & 1).astype(jnp.float32)
                    return accT + jnp.where(lmask, evf * lg, 0.0)

                def slow(accT):
                    niv = (kk + 15) // 16

                    def ivloop(iv, accT):
                        i0 = off + iv * 16
                        di = bd[pl.ds(i0, 16)]
                        pki = bpk[pl.ds(i0, 16)]
                        idxi = pki & IDXMASK
                        imask = (iv * 16 + iota) < kk

                        def jloop(j, Wacc):
                            dj = bd[pl.ds(off + j, 16)][0]
                            pkj = bpk[pl.ds(off + j, 16)][0]
                            exj = bex[pl.ds(off + j, 16)][0]
                            idxj = pkj & IDXMASK
                            lt = jnp.logical_or(
                                dj > di,
                                jnp.logical_and(dj == di, idxj <= idxi))
                            return Wacc + jnp.where(lt, exj, 0.0)
                        Wv = lax.fori_loop(0, kk, jloop, zf)
                        lg = _log16(Wv + Gb + EPS)
                        evf = ((pki >> 30) & 1).astype(jnp.float32)
                        return accT + jnp.where(imask, evf * lg, 0.0)
                    return lax.fori_loop(0, niv, ivloop, accT)

                return accT + (b2[0] - b2[1]).astype(jnp.float32) + Gb

            return lax.fori_loop(0, NBATCH, bucket_body, accT)

        accT = lax.fori_loop(0, NDB, pd_batch, zf)

        v_tf[pl.ds(0, 16)] = accT
        pltpu.sync_copy(v_tf.at[pl.ds(0, 16)], sh_T.at[pl.ds(tid * 16, 16)])
        plsc.subcore_barrier()

        @pl.when(tid == 0)
        def _final():
            pltpu.sync_copy(sh_T, v_tf.at[pl.ds(0, 256)])
            T = zf
            for r in range(NT):
                T = T + v_tf[pl.ds(r * 16, 16)]
            pltpu.sync_copy(sh_se, v_tf.at[pl.ds(0, 256)])
            SE = zf
            for r in range(NT):
                SE = SE + v_tf[pl.ds(r * 16, 16)]
            pltpu.sync_copy(sh_selh, v_tf.at[pl.ds(0, 256)])
            SELH = zf
            for r in range(NT):
                SELH = SELH + v_tf[pl.ds(r * 16, 16)]
            num = zf + (jnp.sum(T) + gamma * jnp.sum(SE) - jnp.sum(SELH))
            den = zf + jnp.sum(SE)
            v_tf[pl.ds(0, 16)] = num / den
            pltpu.sync_copy(v_tf.at[pl.ds(0, 16)], out_hbm)

    out = k(input, target, weight.astype(jnp.float32))
    return out[0]


# speedup vs baseline: 7.2208x; 7.2208x over previous
import functools
import jax
import jax.numpy as jnp
from jax import lax
from jax.experimental import pallas as pl
from jax.experimental.pallas import tpu as pltpu, tpu_sc as plsc

N = 262144
EPS = 1e-7
NT = 16
CHUNK = N // NT
B = 16384
SLICE = B // NT
W = 2048
NW = CHUNK // W
NBATCH = 256
NDB = SLICE // NBATCH
CAP = 5120
CAPBUF = CAP + 32
IDXMASK = (1 << 18) - 1
LN2 = 0.6931471805599453


def _log16(x):
    xi = lax.bitcast_convert_type(x, jnp.int32)
    e = ((xi >> 23) & 0xFF) - 127
    m = lax.bitcast_convert_type((xi & 0x7FFFFF) | 0x3F800000, jnp.float32)
    big = m > 1.4142135
    r = jnp.where(big, m * 0.5, m)
    ef = (e + jnp.where(big, 1, 0)).astype(jnp.float32)
    t = (r - 1.0) / (r + 1.0)
    t2 = t * t
    lnr = 2.0 * t * (1.0 + t2 * (1.0 / 3.0 + t2 * (0.2 + t2 / 7.0)))
    return ef * LN2 + lnr


def _dgather(v, idx):
    return lax.gather(
        v, idx[:, None],
        dimension_numbers=lax.GatherDimensionNumbers(
            offset_dims=(), collapsed_slice_dims=(0,), start_index_map=(0,)),
        slice_sizes=(1,),
        mode=lax.GatherScatterMode.PROMISE_IN_BOUNDS)


def kernel(input, target, weight):
    mesh = plsc.VectorSubcoreMesh(
        core_axis_name="c", subcore_axis_name="s", num_cores=1)

    @functools.partial(
        pl.kernel, mesh=mesh,
        out_type=jax.ShapeDtypeStruct((16,), jnp.float32),
        scratch_types=[
            pltpu.VMEM((B,), jnp.int32),
            pltpu.VMEM((W,), jnp.float32),
            pltpu.VMEM((W,), jnp.float32),
            pltpu.VMEM((W,), jnp.float32),
            pltpu.VMEM((W,), jnp.int32),
            pltpu.VMEM((W,), jnp.float32),
            pltpu.VMEM((W,), jnp.int32),
            pltpu.VMEM((W,), jnp.int32),
            pltpu.VMEM((W,), jnp.int32),
            pltpu.VMEM((CAPBUF,), jnp.float32),
            pltpu.VMEM((CAPBUF,), jnp.int32),
            pltpu.VMEM((CAPBUF,), jnp.float32),
            pltpu.VMEM((SLICE,), jnp.int32),
            pltpu.VMEM((SLICE,), jnp.float32),
            pltpu.VMEM((256,), jnp.float32),
            pltpu.VMEM((256,), jnp.int32),
            pltpu.VMEM((NBATCH + 16,), jnp.int32),
            pltpu.SMEM((NBATCH,), jnp.float32),
            pltpu.SemaphoreType.DMA,
            pltpu.SemaphoreType.DMA,
            pltpu.VMEM_SHARED((NT * B,), jnp.int32),
            pltpu.HBM((B + 16,), jnp.int32),
            pltpu.VMEM_SHARED((B,), jnp.float32),
            pltpu.VMEM_SHARED((N,), jnp.float32),
            pltpu.VMEM_SHARED((N,), jnp.int32),
            pltpu.VMEM_SHARED((N,), jnp.float32),
            pltpu.VMEM_SHARED((NT * 16,), jnp.float32),
            pltpu.VMEM_SHARED((NT * 16,), jnp.float32),
            pltpu.VMEM_SHARED((NT * 16,), jnp.float32),
            pltpu.VMEM_SHARED((NT * 16,), jnp.float32),
            pltpu.VMEM_SHARED((NT * 16,), jnp.int32),
        ],
        compiler_params=pltpu.CompilerParams(needs_layout_passes=False),
    )
    def k(lh_hbm, d_hbm, ev_hbm, out_hbm,
          v_cnt, wd, wlh, wev, wpk, wex, wpos, wbeta, wone,
          bd, bpk, bex, v_sl, v_fl, v_tf, v_ti, vbst, smf, sem, sem_in,
          sh_hist, sh_bstart, sh_bsum, sh_gd, sh_gpk, sh_gex,
          sh_max, sh_se, sh_selh, sh_T, sh_tot):
        tid = lax.axis_index("s")
        iota = lax.iota(jnp.int32, 16)
        zf = jnp.zeros((16,), jnp.float32)
        zi = jnp.zeros((16,), jnp.int32)
        full15 = jnp.full((16,), 15, jnp.int32)

        def stage3(base):
            pltpu.sync_copy(d_hbm.at[pl.ds(base, W)], wd)
            pltpu.sync_copy(lh_hbm.at[pl.ds(base, W)], wlh)
            pltpu.sync_copy(ev_hbm.at[pl.ds(base, W)], wev)

        def _zero_cnt(i, _):
            v_cnt[pl.ds(i * 16, 16)] = zi
            return 0
        lax.fori_loop(0, B // 16, _zero_cnt, 0)

        def _zero_fl(i, _):
            v_fl[pl.ds(i * 16, 16)] = zf
            return 0
        lax.fori_loop(0, SLICE // 16, _zero_fl, 0)
        pltpu.sync_copy(v_fl, sh_bsum.at[pl.ds(tid * SLICE, SLICE)])

        def _init_hist(r, _):
            pltpu.sync_copy(v_cnt.at[pl.ds(r * SLICE, SLICE)],
                            sh_hist.at[pl.ds(tid * B + r * SLICE, SLICE)])
            return 0
        lax.fori_loop(0, NT, _init_hist, 0)
        plsc.subcore_barrier()

        histbase = tid * B

        def pa_win(w, carry):
            mx, se, selh = carry
            stage3(tid * CHUNK + w * W)

            def body(i, c):
                mx, se, selh = c
                d = wd[pl.ds(i * 16, 16)]
                lh = wlh[pl.ds(i * 16, 16)]
                ev = wev[pl.ds(i * 16, 16)]
                beta = (B - 1) - jnp.minimum(
                    (d * (B / 100.0)).astype(jnp.int32), B - 1)
                wbeta[pl.ds(i * 16, 16)] = beta + histbase
                wone[pl.ds(i * 16, 16)] = jnp.ones((16,), jnp.int32)
                return (jnp.maximum(mx, lh), se + ev, selh + ev * lh)
            mx, se, selh = lax.fori_loop(0, W // 16, body, (mx, se, selh))
            pltpu.sync_copy(wone, sh_hist.at[wbeta], add=True)
            return (mx, se, selh)

        mx, se, selh = lax.fori_loop(
            0, NW, pa_win,
            (jnp.full((16,), -3.4e38, jnp.float32), zf, zf))

        v_tf[pl.ds(0, 16)] = mx
        pltpu.sync_copy(v_tf.at[pl.ds(0, 16)], sh_max.at[pl.ds(tid * 16, 16)])
        v_tf[pl.ds(0, 16)] = se
        pltpu.sync_copy(v_tf.at[pl.ds(0, 16)], sh_se.at[pl.ds(tid * 16, 16)])
        v_tf[pl.ds(0, 16)] = selh
        pltpu.sync_copy(v_tf.at[pl.ds(0, 16)], sh_selh.at[pl.ds(tid * 16, 16)])
        plsc.subcore_barrier()

        pltpu.sync_copy(sh_max, v_tf.at[pl.ds(0, 256)])

        def _gmax(r, g):
            return jnp.maximum(g, v_tf[pl.ds(r * 16, 16)])
        gmx = lax.fori_loop(0, NT, _gmax,
                            jnp.full((16,), -3.4e38, jnp.float32))
        gamma = jnp.max(gmx)

        def _stage_cols(r, _):
            pltpu.sync_copy(sh_hist.at[pl.ds(r * B + tid * SLICE, SLICE)],
                            v_cnt.at[pl.ds(r * SLICE, SLICE)])
            return 0
        lax.fori_loop(0, NT, _stage_cols, 0)

        def _tot(i, _):
            acc = zi
            for r in range(NT):
                acc = acc + v_cnt[pl.ds(r * SLICE + i * 16, 16)]
            v_sl[pl.ds(i * 16, 16)] = acc
            return 0
        lax.fori_loop(0, SLICE // 16, _tot, 0)

        def _scan_i(i, carry):
            v = v_sl[pl.ds(i * 16, 16)]
            cs = plsc.cumsum(v) + carry
            v_sl[pl.ds(i * 16, 16)] = cs
            return _dgather(cs, full15)
        carry = lax.fori_loop(0, SLICE // 16, _scan_i, zi)
        v_ti[pl.ds(0, 16)] = carry
        pltpu.sync_copy(v_ti.at[pl.ds(0, 16)], sh_tot.at[pl.ds(tid * 16, 16)])
        plsc.subcore_barrier()

        pltpu.sync_copy(sh_tot, v_ti.at[pl.ds(0, 256)])
        tot16 = plsc.load_gather(v_ti, [iota * 16])
        exc = plsc.cumsum(tot16) - tot16
        mycarry = _dgather(exc, jnp.full((16,), tid, jnp.int32))

        def _excl(i, _):
            incl = v_sl[pl.ds(i * 16, 16)]
            tot = zi
            for r in range(NT):
                tot = tot + v_cnt[pl.ds(r * SLICE + i * 16, 16)]
            v_sl[pl.ds(i * 16, 16)] = incl - tot + mycarry
            return 0
        lax.fori_loop(0, SLICE // 16, _excl, 0)
        pltpu.sync_copy(v_sl, sh_bstart.at[pl.ds(tid * SLICE, SLICE)])

        @pl.when(tid == NT - 1)
        def _sentinel():
            v_ti[pl.ds(16, 16)] = jnp.full((16,), N, jnp.int32)
            pltpu.sync_copy(v_ti.at[pl.ds(16, 16)], sh_bstart.at[pl.ds(B, 16)])

        def _bases(r, _):
            def bl(i, _):
                acc = v_sl[pl.ds(i * 16, 16)]
                h = v_cnt[pl.ds(r * SLICE + i * 16, 16)]
                v_sl[pl.ds(i * 16, 16)] = acc + h
                v_cnt[pl.ds(r * SLICE + i * 16, 16)] = acc
                return 0
            lax.fori_loop(0, SLICE // 16, bl, 0)
            pltpu.sync_copy(v_cnt.at[pl.ds(r * SLICE, SLICE)],
                            sh_hist.at[pl.ds(r * B + tid * SLICE, SLICE)])
            return 0
        lax.fori_loop(0, NT, _bases, 0)
        plsc.subcore_barrier()

        pltpu.sync_copy(sh_hist.at[pl.ds(tid * B, B)], v_cnt)
        plsc.subcore_barrier()

        def pc_win(w, _):
            cbase = tid * CHUNK + w * W
            stage3(cbase)

            def body(i, _):
                j = i * 16
                d = wd[pl.ds(j, 16)]
                lh = wlh[pl.ds(j, 16)]
                ev = wev[pl.ds(j, 16)]
                beta = (B - 1) - jnp.minimum(
                    (d * (B / 100.0)).astype(jnp.int32), B - 1)
                ex = jnp.exp(lh - gamma)
                gidx = cbase + j + iota
                pk = gidx | (ev.astype(jnp.int32) << 30)
                sb, sl = plsc.sort_key_val(beta, iota)
                prev = _dgather(sb, jnp.maximum(iota - 1, 0))
                eqp = jnp.logical_and(sb == prev, iota > 0)
                rstart = plsc.cummax(jnp.where(eqp, 0, iota))
                rpos = iota - rstart
                nxt = _dgather(sb, jnp.minimum(iota + 1, 15))
                lasts = jnp.logical_or(sb != nxt, iota == 15)
                cur = plsc.load_gather(v_cnt, [sb])
                pos = cur + rpos
                plsc.store_scatter(v_cnt, [sb], cur + rpos + 1, mask=lasts)
                wpos[pl.ds(j, 16)] = pos
                wbeta[pl.ds(j, 16)] = sb
                wd[pl.ds(j, 16)] = _dgather(d, sl)
                wpk[pl.ds(j, 16)] = _dgather(pk, sl)
                wex[pl.ds(j, 16)] = _dgather(ex, sl)
                return 0
            lax.fori_loop(0, W // 16, body, 0)
            pltpu.sync_copy(wd, sh_gd.at[wpos])
            pltpu.sync_copy(wpk, sh_gpk.at[wpos])
            pltpu.sync_copy(wex, sh_gex.at[wpos])
            pltpu.sync_copy(wex, sh_bsum.at[wbeta], add=True)
            return 0
        lax.fori_loop(0, NW, pc_win, 0)
        plsc.subcore_barrier()

        pltpu.sync_copy(sh_bsum.at[pl.ds(tid * SLICE, SLICE)], v_fl)

        def _fscan(i, carry):
            v = v_fl[pl.ds(i * 16, 16)]
            cs = plsc.cumsum(v) + carry
            v_fl[pl.ds(i * 16, 16)] = cs - v
            return _dgather(cs, full15)
        fcarry = lax.fori_loop(0, SLICE // 16, _fscan, zf)
        v_tf[pl.ds(0, 16)] = fcarry
        pltpu.sync_copy(v_tf.at[pl.ds(0, 16)], sh_T.at[pl.ds(tid * 16, 16)])
        plsc.subcore_barrier()
        pltpu.sync_copy(sh_T, v_tf.at[pl.ds(0, 256)])
        ftot16 = plsc.load_gather(v_tf, [iota * 16])
        fexc = plsc.cumsum(ftot16) - ftot16
        myf = _dgather(fexc, jnp.full((16,), tid, jnp.int32))

        def _fadd(i, _):
            v_fl[pl.ds(i * 16, 16)] = v_fl[pl.ds(i * 16, 16)] + myf
            return 0
        lax.fori_loop(0, SLICE // 16, _fadd, 0)
        pltpu.sync_copy(v_fl, sh_bsum.at[pl.ds(tid * SLICE, SLICE)])
        plsc.subcore_barrier()

        def pd_batch(db, accT):
            bb = tid * SLICE + db * NBATCH
            pltpu.sync_copy(sh_bstart.at[pl.ds(bb, NBATCH + 16)], vbst)
            pltpu.sync_copy(sh_bsum.at[pl.ds(bb, NBATCH)], smf)
            s0 = vbst[pl.ds(0, 16)][0]
            s8 = (s0 // 8) * 8
            start = jnp.minimum(s8, N - CAP)
            pltpu.sync_copy(sh_gd.at[pl.ds(start, CAP)], bd.at[pl.ds(0, CAP)])
            pltpu.sync_copy(sh_gpk.at[pl.ds(start, CAP)],
                            bpk.at[pl.ds(0, CAP)])
            pltpu.sync_copy(sh_gex.at[pl.ds(start, CAP)],
                            bex.at[pl.ds(0, CAP)])

            def bucket_body(b, accT):
                b2 = vbst[pl.ds(b, 16)]
                sb_ = b2[0]
                kk = b2[1] - sb_
                off = sb_ - start
                Gb = smf[b]

                def fast(accT):
                    d16 = bd[pl.ds(off, 16)]
                    pk16 = bpk[pl.ds(off, 16)]
                    ex16 = bex[pl.ds(off, 16)]
                    lmask = iota < kk
                    sd, sv, _m1 = plsc.sort_key_val(
                        d16, iota, mask=lmask, descending=True)
                    svc = jnp.where(lmask, sv, 0)
                    spk = _dgather(pk16, svc)
                    sex = _dgather(ex16, svc)
                    prev = _dgather(sd, jnp.maximum(iota - 1, 0))
                    eqp = jnp.logical_and(sd == prev, iota > 0)
                    rstart = plsc.cummax(jnp.where(eqp, 0, iota))
                    key2 = (rstart << 18) | (spk & IDXMASK)
                    key2 = jnp.where(lmask, key2, 0x7FFFFFFF)
                    _k2, sv2 = plsc.sort_key_val(key2, iota)
                    pkf = _dgather(spk, sv2)
                    exf = _dgather(sex, sv2)
                    exm = jnp.where(lmask, exf, 0.0)
                    S = plsc.cumsum(exm) + Gb
                    lg = _log16(S + EPS)
                    evf = ((pkf >> 30) & 1).astype(jnp.float32)
                    return accT + jnp.where(lmask, evf * lg, 0.0)

                def slow(accT):
                    niv = (kk + 15) // 16

                    def ivloop(iv, accT):
                        i0 = off + iv * 16
                        di = bd[pl.ds(i0, 16)]
                        pki = bpk[pl.ds(i0, 16)]
                        idxi = pki & IDXMASK
                        imask = (iv * 16 + iota) < kk

                        def jloop(j, Wacc):
                            dj = bd[pl.ds(off + j, 16)][0]
                            pkj = bpk[pl.ds(off + j, 16)][0]
                            exj = bex[pl.ds(off + j, 16)][0]
                            idxj = pkj & IDXMASK
                            lt = jnp.logical_or(
                                dj > di,
                                jnp.logical_and(dj == di, idxj <= idxi))
                            return Wacc + jnp.where(lt, exj, 0.0)
                        Wv = lax.fori_loop(0, kk, jloop, zf)
                        lg = _log16(Wv + Gb + EPS)
                        evf = ((pki >> 30) & 1).astype(jnp.float32)
                        return accT + jnp.where(imask, evf * lg, 0.0)
                    return lax.fori_loop(0, niv, ivloop, accT)

                return lax.cond(kk <= 16, fast, slow, accT)

            return lax.fori_loop(0, NBATCH, bucket_body, accT)

        accT = lax.fori_loop(0, NDB, pd_batch, zf)

        v_tf[pl.ds(0, 16)] = accT
        pltpu.sync_copy(v_tf.at[pl.ds(0, 16)], sh_T.at[pl.ds(tid * 16, 16)])
        plsc.subcore_barrier()

        @pl.when(tid == 0)
        def _final():
            pltpu.sync_copy(sh_T, v_tf.at[pl.ds(0, 256)])
            T = zf
            for r in range(NT):
                T = T + v_tf[pl.ds(r * 16, 16)]
            pltpu.sync_copy(sh_se, v_tf.at[pl.ds(0, 256)])
            SE = zf
            for r in range(NT):
                SE = SE + v_tf[pl.ds(r * 16, 16)]
            pltpu.sync_copy(sh_selh, v_tf.at[pl.ds(0, 256)])
            SELH = zf
            for r in range(NT):
                SELH = SELH + v_tf[pl.ds(r * 16, 16)]
            num = zf + (jnp.sum(T) + gamma * jnp.sum(SE) - jnp.sum(SELH))
            den = zf + jnp.sum(SE)
            v_tf[pl.ds(0, 16)] = num / den
            pltpu.sync_copy(v_tf.at[pl.ds(0, 16)], out_hbm)

    out = k(input, target, weight.astype(jnp.float32))
    return out[0]
